# baseline (speedup 1.0000x reference)
import functools

import jax
import jax.numpy as jnp
from jax import lax
from jax.experimental import pallas as pl
from jax.experimental.pallas import tpu as pltpu
from jax.experimental.pallas import tpu_sc as plsc

_NUM_FIELDS = 26
_SLOTS = 32
_VOCAB = 10000
_COMPACT_ROWS = _NUM_FIELDS * _VOCAB
_D = 16
_BATCH = 4096
_N_IDX = _BATCH * _SLOTS
_WINDOW = 128


def _gather_kernel(table_hbm, idx_hbm, out_hbm):
    def body(i_vmem, o_vmem):
        pltpu.sync_copy(table_hbm.at[i_vmem.at[0]], o_vmem)

    pltpu.emit_pipeline(
        body,
        grid=(_N_IDX // _WINDOW,),
        in_specs=[pl.BlockSpec((1, _WINDOW), lambda i: (0, i))],
        out_specs=[pl.BlockSpec((_WINDOW, _D), lambda i: (i, 0))],
        core_axis_name=("core", "subcore"),
        dimension_semantics=(pltpu.PARALLEL,),
    )(idx_hbm, out_hbm)


@jax.jit
def _sc_gather(table, idx):
    mesh = plsc.VectorSubcoreMesh(core_axis_name="core", subcore_axis_name="subcore")
    k = pl.kernel(
        _gather_kernel,
        out_type=jax.ShapeDtypeStruct((_N_IDX, _D), jnp.float32),
        mesh=mesh,
        compiler_params=pltpu.CompilerParams(use_tc_tiling_on_sc=False),
    )
    return k(table, idx)


def _tc_body(H_ref, M_ref, d_ref, W1_ref, b1_ref, g1_ref, be1_ref,
             W2_ref, b2_ref, g2_ref, be2_ref, W3_ref, b3_ref, out_ref):
    H = H_ref[...]
    G = jnp.dot(H, M_ref[...], preferred_element_type=jnp.float32)
    quad = jnp.sum(G * H, axis=1, keepdims=True)
    diag = jnp.sum(H * H * d_ref[...], axis=1, keepdims=True)
    fwfm = 0.5 * (quad - diag)
    h = jnp.dot(H, W1_ref[...], preferred_element_type=jnp.float32) + b1_ref[...]
    m = jnp.mean(h, axis=0, keepdims=True)
    v = jnp.mean((h - m) * (h - m), axis=0, keepdims=True)
    h = jnp.maximum(g1_ref[...] * (h - m) * lax.rsqrt(v + 1e-5) + be1_ref[...], 0.0)
    h = jnp.dot(h, W2_ref[...], preferred_element_type=jnp.float32) + b2_ref[...]
    m = jnp.mean(h, axis=0, keepdims=True)
    v = jnp.mean((h - m) * (h - m), axis=0, keepdims=True)
    h = jnp.maximum(g2_ref[...] * (h - m) * lax.rsqrt(v + 1e-5) + be2_ref[...], 0.0)
    o = jnp.dot(h, W3_ref[...], preferred_element_type=jnp.float32) + b3_ref[...]
    out_ref[...] = jax.nn.sigmoid(fwfm + o)


def kernel(x, emb_tables, field_cov_w, W1, b1, gamma1, beta1,
           W2, b2, gamma2, beta2, W3, b3):
    compact = jnp.concatenate(
        [lax.slice(emb_tables, (i, _VOCAB * i, 0), (i + 1, _VOCAB * (i + 1), _D))
         for i in range(_NUM_FIELDS)], axis=1)[0]

    idx_f = x + _VOCAB * jnp.arange(_NUM_FIELDS, dtype=x.dtype)[None, :]
    n = jnp.arange(_BATCH, dtype=x.dtype)[:, None] * jnp.ones((1, _SLOTS - _NUM_FIELDS), x.dtype)
    dummy = (n * _SLOTS) % _COMPACT_ROWS
    idx2 = jnp.concatenate([idx_f, dummy], axis=1)
    idx = idx2.reshape(_BATCH // 8, 8, 4, 8).transpose(0, 2, 1, 3).reshape(1, _N_IDX)

    rows = _sc_gather(compact, idx)
    H = (rows.reshape(_BATCH // 8, 4, 8, 8, _D)
             .transpose(0, 2, 1, 3, 4)
             .reshape(_BATCH, _SLOTS * _D))

    sym = (field_cov_w.T + field_cov_w) * 0.5
    M = jnp.kron(sym, jnp.eye(_D, dtype=jnp.float32))
    Mp = jnp.pad(M, ((0, 96), (0, 96)))
    d = jnp.pad(jnp.repeat(jnp.diagonal(sym), _D), (0, 96)).reshape(1, -1)
    W1p = jnp.pad(W1, ((0, 96), (0, 0)))

    out = pl.pallas_call(
        _tc_body,
        out_shape=jax.ShapeDtypeStruct((_BATCH, 1), jnp.float32),
    )(H, Mp, d,
      W1p, b1.reshape(1, -1), gamma1.reshape(1, -1), beta1.reshape(1, -1),
      W2, b2.reshape(1, -1), gamma2.reshape(1, -1), beta2.reshape(1, -1),
      W3, b3.reshape(1, -1))
    return out.reshape(_BATCH)


# speedup vs baseline: 1.4518x; 1.4518x over previous
import functools

import jax
import jax.numpy as jnp
from jax import lax
from jax.experimental import pallas as pl
from jax.experimental.pallas import tpu as pltpu
from jax.experimental.pallas import tpu_sc as plsc

_NUM_FIELDS = 26
_SLOTS = 32
_VOCAB = 10000
_COMPACT_ROWS = _NUM_FIELDS * _VOCAB
_D = 16
_BATCH = 4096
_N_IDX = _BATCH * _SLOTS
_WINDOW = 128


def _gather_kernel(table_hbm, idx_hbm, out_hbm):
    def body(i_vmem, o_vmem):
        pltpu.sync_copy(table_hbm.at[i_vmem.at[0]], o_vmem)

    pltpu.emit_pipeline(
        body,
        grid=(_N_IDX // _WINDOW,),
        in_specs=[pl.BlockSpec((1, _WINDOW), lambda i: (0, i))],
        out_specs=[pl.BlockSpec((_WINDOW, _D), lambda i: (i, 0))],
        core_axis_name=("core", "subcore"),
        dimension_semantics=(pltpu.PARALLEL,),
    )(idx_hbm, out_hbm)


@jax.jit
def _sc_gather(table, idx):
    mesh = plsc.VectorSubcoreMesh(core_axis_name="core", subcore_axis_name="subcore")
    k = pl.kernel(
        _gather_kernel,
        out_type=jax.ShapeDtypeStruct((_N_IDX, _D), jnp.float32),
        mesh=mesh,
        compiler_params=pltpu.CompilerParams(use_tc_tiling_on_sc=False),
    )
    return k(table, idx)


def _tc_body(H_ref, M_ref, d_ref, W1_ref, b1_ref, g1_ref, be1_ref,
             W2_ref, b2_ref, g2_ref, be2_ref, W3_ref, b3_ref, out_ref):
    H = jnp.reshape(H_ref[...], (_BATCH, _SLOTS * _D))
    G = jnp.dot(H, M_ref[...], preferred_element_type=jnp.float32)
    quad = jnp.sum(G * H, axis=1, keepdims=True)
    diag = jnp.sum(H * H * d_ref[...], axis=1, keepdims=True)
    fwfm = 0.5 * (quad - diag)
    h = jnp.dot(H, W1_ref[...], preferred_element_type=jnp.float32) + b1_ref[...]
    m = jnp.mean(h, axis=0, keepdims=True)
    v = jnp.mean((h - m) * (h - m), axis=0, keepdims=True)
    h = jnp.maximum(g1_ref[...] * (h - m) * lax.rsqrt(v + 1e-5) + be1_ref[...], 0.0)
    h = jnp.dot(h, W2_ref[...], preferred_element_type=jnp.float32) + b2_ref[...]
    m = jnp.mean(h, axis=0, keepdims=True)
    v = jnp.mean((h - m) * (h - m), axis=0, keepdims=True)
    h = jnp.maximum(g2_ref[...] * (h - m) * lax.rsqrt(v + 1e-5) + be2_ref[...], 0.0)
    o = jnp.dot(h, W3_ref[...], preferred_element_type=jnp.float32) + b3_ref[...]
    out_ref[...] = jax.nn.sigmoid(fwfm + o)


def kernel(x, emb_tables, field_cov_w, W1, b1, gamma1, beta1,
           W2, b2, gamma2, beta2, W3, b3):
    compact = jnp.concatenate(
        [lax.slice(emb_tables, (i, _VOCAB * i, 0), (i + 1, _VOCAB * (i + 1), _D))
         for i in range(_NUM_FIELDS)], axis=1)[0]

    idx_f = x + _VOCAB * jnp.arange(_NUM_FIELDS, dtype=x.dtype)[None, :]
    n = jnp.arange(_BATCH, dtype=x.dtype)[:, None] * jnp.ones((1, _SLOTS - _NUM_FIELDS), x.dtype)
    dummy = (n * _SLOTS) % _COMPACT_ROWS
    idx = jnp.concatenate([idx_f, dummy], axis=1).reshape(1, _N_IDX)

    rows = _sc_gather(compact, idx)
    H = rows.reshape(_BATCH * 4, _D * 8)

    sym = (field_cov_w.T + field_cov_w) * 0.5
    M = jnp.kron(sym, jnp.eye(_D, dtype=jnp.float32))
    Mp = jnp.pad(M, ((0, 96), (0, 96)))
    d = jnp.pad(jnp.repeat(jnp.diagonal(sym), _D), (0, 96)).reshape(1, -1)
    W1p = jnp.pad(W1, ((0, 96), (0, 0)))

    out = pl.pallas_call(
        _tc_body,
        out_shape=jax.ShapeDtypeStruct((_BATCH, 1), jnp.float32),
    )(H, Mp, d,
      W1p, b1.reshape(1, -1), gamma1.reshape(1, -1), beta1.reshape(1, -1),
      W2, b2.reshape(1, -1), gamma2.reshape(1, -1), beta2.reshape(1, -1),
      W3, b3.reshape(1, -1))
    return out.reshape(_BATCH)


# speedup vs baseline: 2.8876x; 1.9890x over previous
import functools

import jax
import jax.numpy as jnp
from jax import lax
from jax.experimental import pallas as pl
from jax.experimental.pallas import tpu as pltpu
from jax.experimental.pallas import tpu_sc as plsc

_NUM_FIELDS = 26
_SLOTS = 32
_VOCAB = 10000
_VOCAB_PAD = 10240
_COMPACT_ROWS = _NUM_FIELDS * _VOCAB_PAD
_D = 16
_BATCH = 4096
_N_IDX = _BATCH * _SLOTS
_WINDOW = 128


_CHUNK = _VOCAB_PAD // 8


_WIN = _VOCAB + 112
_MAX_START = (26 * _VOCAB - _WIN) // 128 * 128


def _pack_start(f):
    return min(f * _VOCAB // 128 * 128, _MAX_START)


def _pack_body(tab_ref, tail_ref, out_ref, scratch, sem):
    f = pl.program_id(0)

    def dma(g, slot):
        start = pl.multiple_of(
            jnp.minimum(g * _VOCAB // 128 * 128, _MAX_START), 128)
        return pltpu.make_async_copy(
            tab_ref.at[g, :, pl.ds(start, _WIN)],
            scratch.at[slot, :, pl.ds(0, _WIN)],
            sem.at[slot])

    @pl.when(f == 0)
    def _():
        dma(0, 0).start()

    @pl.when(f + 1 < _NUM_FIELDS)
    def _():
        dma(f + 1, (f + 1) % 2).start()

    dma(f, f % 2).wait()
    for q in range(8):
        out_ref[0, :, 16 * q:16 * (q + 1)] = jnp.transpose(
            scratch[f % 2, :, _CHUNK * q:_CHUNK * (q + 1)])

    @pl.when(f == _NUM_FIELDS - 1)
    def _():
        out_ref[0, pl.ds(1152, 32), 112:128] = jnp.transpose(tail_ref[...])


def _gather_kernel(table_hbm, idx_hbm, out_hbm):
    def body(i_vmem, o_vmem):
        pltpu.sync_copy(table_hbm.at[i_vmem.at[0]], o_vmem)

    pltpu.emit_pipeline(
        body,
        grid=(_N_IDX // _WINDOW,),
        in_specs=[pl.BlockSpec((1, _WINDOW), lambda i: (0, i))],
        out_specs=[pl.BlockSpec((_WINDOW, _D), lambda i: (i, 0))],
        core_axis_name=("core", "subcore"),
        dimension_semantics=(pltpu.PARALLEL,),
    )(idx_hbm, out_hbm)


@jax.jit
def _sc_gather(table, idx):
    mesh = plsc.VectorSubcoreMesh(core_axis_name="core", subcore_axis_name="subcore")
    k = pl.kernel(
        _gather_kernel,
        out_type=jax.ShapeDtypeStruct((_N_IDX, _D), jnp.float32),
        mesh=mesh,
        compiler_params=pltpu.CompilerParams(use_tc_tiling_on_sc=False),
    )
    return k(table, idx)


def _tc_body(H_ref, M_ref, d_ref, W1_ref, b1_ref, g1_ref, be1_ref,
             W2_ref, b2_ref, g2_ref, be2_ref, W3_ref, b3_ref, out_ref):
    H = jnp.reshape(H_ref[...], (_BATCH, _SLOTS * _D))
    G = jnp.dot(H, M_ref[...], preferred_element_type=jnp.float32)
    quad = jnp.sum(G * H, axis=1, keepdims=True)
    diag = jnp.sum(H * H * d_ref[...], axis=1, keepdims=True)
    fwfm = 0.5 * (quad - diag)
    h = jnp.dot(H, W1_ref[...], preferred_element_type=jnp.float32) + b1_ref[...]
    m = jnp.mean(h, axis=0, keepdims=True)
    v = jnp.mean((h - m) * (h - m), axis=0, keepdims=True)
    h = jnp.maximum(g1_ref[...] * (h - m) * lax.rsqrt(v + 1e-5) + be1_ref[...], 0.0)
    h = jnp.dot(h, W2_ref[...], preferred_element_type=jnp.float32) + b2_ref[...]
    m = jnp.mean(h, axis=0, keepdims=True)
    v = jnp.mean((h - m) * (h - m), axis=0, keepdims=True)
    h = jnp.maximum(g2_ref[...] * (h - m) * lax.rsqrt(v + 1e-5) + be2_ref[...], 0.0)
    o = jnp.dot(h, W3_ref[...], preferred_element_type=jnp.float32) + b3_ref[...]
    out_ref[...] = jax.nn.sigmoid(fwfm + o)


def kernel(x, emb_tables, field_cov_w, W1, b1, gamma1, beta1,
           W2, b2, gamma2, beta2, W3, b3):
    tab_t = jnp.transpose(emb_tables, (0, 2, 1))
    tail_t = tab_t[_NUM_FIELDS - 1, :, _NUM_FIELDS * _VOCAB - 32:]
    packed = pl.pallas_call(
        _pack_body,
        grid=(_NUM_FIELDS,),
        in_specs=[pl.BlockSpec(memory_space=pl.ANY),
                  pl.BlockSpec((_D, 32), lambda f: (0, 0))],
        out_specs=pl.BlockSpec((1, _VOCAB_PAD * _D // 128, 128), lambda f: (f, 0, 0)),
        out_shape=jax.ShapeDtypeStruct((_NUM_FIELDS, _VOCAB_PAD * _D // 128, 128), jnp.float32),
        scratch_shapes=[pltpu.VMEM((2, _D, _VOCAB_PAD), jnp.float32),
                        pltpu.SemaphoreType.DMA((2,))],
    )(tab_t, tail_t)
    compact = packed.reshape(_COMPACT_ROWS, _D)

    shift = jnp.asarray([_VOCAB * f - _pack_start(f) for f in range(_NUM_FIELDS)], x.dtype)
    s = x + shift[None, :]
    fbase = _VOCAB_PAD * jnp.arange(_NUM_FIELDS, dtype=x.dtype)[None, :]
    idx_f = fbase + 8 * (s % _CHUNK) + s // _CHUNK
    n = jnp.arange(_BATCH, dtype=x.dtype)[:, None] * jnp.ones((1, _SLOTS - _NUM_FIELDS), x.dtype)
    sd = (n * _SLOTS) % _VOCAB
    dummy = 8 * (sd % _CHUNK) + sd // _CHUNK
    idx = jnp.concatenate([idx_f, dummy], axis=1).reshape(1, _N_IDX)

    rows = _sc_gather(compact, idx)
    H = rows.reshape(_BATCH * 4, _D * 8)

    sym = (field_cov_w.T + field_cov_w) * 0.5
    M = jnp.kron(sym, jnp.eye(_D, dtype=jnp.float32))
    Mp = jnp.pad(M, ((0, 96), (0, 96)))
    d = jnp.pad(jnp.repeat(jnp.diagonal(sym), _D), (0, 96)).reshape(1, -1)
    W1p = jnp.pad(W1, ((0, 96), (0, 0)))

    out = pl.pallas_call(
        _tc_body,
        out_shape=jax.ShapeDtypeStruct((_BATCH, 1), jnp.float32),
    )(H, Mp, d,
      W1p, b1.reshape(1, -1), gamma1.reshape(1, -1), beta1.reshape(1, -1),
      W2, b2.reshape(1, -1), gamma2.reshape(1, -1), beta2.reshape(1, -1),
      W3, b3.reshape(1, -1))
    return out.reshape(_BATCH)
